# baseline (speedup 1.0000x reference)
import functools
import math

import jax
import jax.numpy as jnp
from jax import lax
from jax.experimental import pallas as pl
from jax.experimental.pallas import tpu as pltpu
from jax.experimental.pallas import tpu_sc as plsc

_VOCAB = 100000
_D = 1024
_PAD = 1
_EPS = 1e-05
_EPS_FOLDED = _EPS / float(_D)

_NC = 2
_NS = 16
_L = 16
_NW = _NC * _NS
_B = 4 * 8192
_TPW = _B // _NW
_T = 32
_NCHUNK = _TPW // _T


def _rsqrt_vec(x):
    i = plsc.bitcast(x, jnp.int32)
    i = jnp.int32(0x5F3759DF) - lax.shift_right_arithmetic(i, 1)
    y = plsc.bitcast(i, jnp.float32)
    for _ in range(3):
        y = y * (1.5 - 0.5 * x * y * y)
    return y


_MESH = plsc.VectorSubcoreMesh(core_axis_name="c", subcore_axis_name="s")


@functools.partial(
    pl.kernel,
    out_type=(
        jax.ShapeDtypeStruct((_B, _D), jnp.float32),
        jax.ShapeDtypeStruct((_B,), jnp.int32),
    ),
    mesh=_MESH,
    scratch_types=[
        pltpu.VMEM((_TPW,), jnp.int32),
        pltpu.VMEM((_T, _D), jnp.float32),
        pltpu.VMEM((_D,), jnp.float32),
        pltpu.VMEM((_D,), jnp.float32),
        pltpu.VMEM((_TPW,), jnp.int32),
        pltpu.SemaphoreType.DMA,
    ],
    compiler_params=pltpu.CompilerParams(use_tc_tiling_on_sc=False,
                                         needs_layout_passes=False),
)
def _frontend(idx_hbm, table_hbm, gamma_hbm, beta_hbm, out_hbm, mask_hbm,
              idx_v, rows, gamma_v, beta_v, mask_v, sem):
    wid = lax.axis_index("s") * _NC + lax.axis_index("c")
    base = wid * _TPW

    pltpu.sync_copy(idx_hbm.at[pl.ds(base, _TPW)], idx_v)
    pltpu.sync_copy(gamma_hbm, gamma_v)
    pltpu.sync_copy(beta_hbm, beta_v)

    def mask_body(i, carry):
        v = idx_v[pl.ds(i * _L, _L)]
        mask_v[pl.ds(i * _L, _L)] = jnp.where(v == _PAD, jnp.int32(1),
                                              jnp.int32(0))
        return carry
    lax.fori_loop(0, _TPW // _L, mask_body, 0)
    pltpu.sync_copy(mask_v, mask_hbm.at[pl.ds(base, _TPW)])

    zeros = jnp.zeros((_L,), jnp.float32)
    izeros = jnp.zeros((_L,), jnp.int32)

    def chunk_body(c, carry):
        pltpu.async_copy(table_hbm.at[idx_v.at[pl.ds(c * _T, _T)]],
                         rows, sem).wait()

        for grp in range(_T // _L):
            tokvec = lax.iota(jnp.int32, _L) + (grp * _L)

            def p1(p, carry2):
                pvec, s, s2 = carry2
                v = plsc.load_gather(rows, [tokvec, pvec])
                return (pvec + 1, s + v, s2 + v * v)
            _, s, s2 = lax.fori_loop(0, _D, p1, (izeros, zeros, zeros),
                                     unroll=8)
            mean = s * (1.0 / _D)
            var = s2 * (1.0 / _D) - mean * mean
            rinv = _rsqrt_vec(var + _EPS_FOLDED)
            mr = mean * rinv

            lane_consts = [jnp.full((_L,), i, jnp.int32) for i in range(_L)]

            def p2(j, carry2):
                pvec = carry2
                gv = gamma_v[pl.ds(j * _L, _L)]
                bv = beta_v[pl.ds(j * _L, _L)]
                for i in range(_L):
                    gs = gv.at[lane_consts[i]].get(mode="promise_in_bounds")
                    bs = bv.at[lane_consts[i]].get(mode="promise_in_bounds")
                    v = plsc.load_gather(rows, [tokvec, pvec])
                    out = (v * rinv - mr) * gs + bs
                    plsc.store_scatter(rows, [tokvec, pvec], out)
                    pvec = pvec + 1
                return pvec
            lax.fori_loop(0, _D // _L, p2, izeros)

        pltpu.sync_copy(rows, out_hbm.at[pl.ds(base + c * _T, _T)])
        return carry
    lax.fori_loop(0, _NCHUNK, chunk_body, 0)


def kernel(token_indices, embed_table, gamma, beta):
    idx_flat = token_indices.reshape(-1)
    out, mask = _frontend(idx_flat, embed_table, gamma, beta)
    embeds = out.reshape(token_indices.shape + (_D,))
    padding_mask = mask.reshape(token_indices.shape).astype(jnp.bool_)
    return embeds, padding_mask


# speedup vs baseline: 1.6064x; 1.6064x over previous
import functools

import jax
import jax.numpy as jnp
from jax import lax
from jax.experimental import pallas as pl
from jax.experimental.pallas import tpu as pltpu
from jax.experimental.pallas import tpu_sc as plsc

_D = 1024
_PAD = 1
_EPS = 1e-05
_EPS_FOLDED = _EPS / float(_D)

_NC = 2
_NS = 16
_L = 16
_NW = _NC * _NS
_B = 4 * 8192
_TPW = _B // _NW
_T = 16
_NCHUNK = _TPW // _T
_UNROLL = 8


def _rsqrt_vec(x):
    i = plsc.bitcast(x, jnp.int32)
    i = jnp.int32(0x5F3759DF) - lax.shift_right_arithmetic(i, 1)
    y = plsc.bitcast(i, jnp.float32)
    for _ in range(3):
        y = y * (1.5 - 0.5 * x * y * y)
    return y


_MESH = plsc.VectorSubcoreMesh(core_axis_name="c", subcore_axis_name="s")


@functools.partial(
    pl.kernel,
    out_type=(
        jax.ShapeDtypeStruct((_B, _D), jnp.float32),
        jax.ShapeDtypeStruct((_B,), jnp.int32),
    ),
    mesh=_MESH,
    scratch_types=[
        pltpu.VMEM((_TPW,), jnp.int32),
        pltpu.VMEM((_T, _D), jnp.float32),
        pltpu.VMEM((_T, _D), jnp.float32),
        pltpu.VMEM((_T, _D), jnp.float32),
        pltpu.VMEM((_T, _D), jnp.float32),
        pltpu.VMEM((_D,), jnp.float32),
        pltpu.VMEM((_D,), jnp.float32),
        pltpu.VMEM((_TPW,), jnp.int32),
        pltpu.SemaphoreType.DMA,
        pltpu.SemaphoreType.DMA,
        pltpu.SemaphoreType.DMA,
        pltpu.SemaphoreType.DMA,
    ],
    compiler_params=pltpu.CompilerParams(use_tc_tiling_on_sc=False,
                                         needs_layout_passes=False),
)
def _frontend(idx_hbm, table_hbm, gamma_hbm, beta_hbm, out_hbm, mask_hbm,
              idx_v, rows_a, rows_b, sta_a, sta_b, gamma_v, beta_v, mask_v,
              gsem_a, gsem_b, osem_a, osem_b):
    wid = lax.axis_index("s") * _NC + lax.axis_index("c")
    base = wid * _TPW

    pltpu.sync_copy(idx_hbm.at[pl.ds(base, _TPW)], idx_v)
    pltpu.sync_copy(gamma_hbm, gamma_v)
    pltpu.sync_copy(beta_hbm, beta_v)

    def mask_body(i, carry):
        v = idx_v[pl.ds(i * _L, _L)]
        mask_v[pl.ds(i * _L, _L)] = jnp.where(v == _PAD, jnp.int32(1),
                                              jnp.int32(0))
        return carry
    lax.fori_loop(0, _TPW // _L, mask_body, 0)
    pltpu.sync_copy(mask_v, mask_hbm.at[pl.ds(base, _TPW)])

    zeros = jnp.zeros((_L,), jnp.float32)
    z16 = jnp.zeros((_L,), jnp.int32)
    tokflat = lax.iota(jnp.int32, _L) * _D
    lane_consts = [jnp.full((_L,), i, jnp.int32) for i in range(_L)]

    def start_gather(c, rows, gsem):
        pltpu.async_copy(table_hbm.at[idx_v.at[pl.ds(c * _T, _T)]],
                         rows, gsem)

    def wait_gather(rows, gsem):
        pltpu.make_async_copy(table_hbm.at[pl.ds(0, _T)], rows, gsem).wait()

    def start_out(c, sta, osem):
        pltpu.async_copy(sta, out_hbm.at[pl.ds(base + c * _T, _T)], osem)

    def wait_out(sta, osem):
        pltpu.make_async_copy(sta, out_hbm.at[pl.ds(base, _T)], osem).wait()

    def compute(rows, sta):
        nacc = _UNROLL
        init = (tokflat,) + tuple(zeros for _ in range(2 * nacc))
        def p1(p, carry):
            pvec = carry[0]
            accs = list(carry[1:])
            for k in range(_UNROLL):
                v = plsc.load_gather(rows, [z16, pvec + k])
                accs[k] = accs[k] + v
                accs[nacc + k] = accs[nacc + k] + v * v
            return (pvec + _UNROLL,) + tuple(accs)
        fin = plsc.parallel_loop(0, _D, _UNROLL, carry=init)(p1)
        accs = fin[1:]
        s = accs[0]
        q = accs[nacc]
        for k in range(1, nacc):
            s = s + accs[k]
            q = q + accs[nacc + k]
        mean = s * (1.0 / _D)
        var = q * (1.0 / _D) - mean * mean
        rinv = _rsqrt_vec(var + _EPS_FOLDED)
        mr = mean * rinv

        def p2(j, pvec):
            gv = gamma_v[pl.ds(j * _L, _L)]
            bv = beta_v[pl.ds(j * _L, _L)]
            for i in range(_L):
                gs = gv.at[lane_consts[i]].get(mode="promise_in_bounds")
                bs = bv.at[lane_consts[i]].get(mode="promise_in_bounds")
                v = plsc.load_gather(rows, [z16, pvec])
                plsc.store_scatter(sta, [z16, pvec], (v * rinv - mr) * gs + bs)
                pvec = pvec + 1
            return pvec
        plsc.parallel_loop(0, _D // _L, 1, carry=tokflat)(p2)

    start_gather(0, rows_a, gsem_a)

    def pair_body(cp, carry):
        c0 = 2 * cp
        c1 = c0 + 1
        start_gather(c1, rows_b, gsem_b)
        wait_gather(rows_a, gsem_a)

        @pl.when(cp > 0)
        def _():
            wait_out(sta_a, osem_a)
        compute(rows_a, sta_a)
        start_out(c0, sta_a, osem_a)

        @pl.when(cp < _NCHUNK // 2 - 1)
        def _():
            start_gather(c0 + 2, rows_a, gsem_a)
        wait_gather(rows_b, gsem_b)

        @pl.when(cp > 0)
        def _():
            wait_out(sta_b, osem_b)
        compute(rows_b, sta_b)
        start_out(c1, sta_b, osem_b)
        return carry
    lax.fori_loop(0, _NCHUNK // 2, pair_body, 0)

    wait_out(sta_a, osem_a)
    wait_out(sta_b, osem_b)


def kernel(token_indices, embed_table, gamma, beta):
    idx_flat = token_indices.reshape(-1)
    out, mask = _frontend(idx_flat, embed_table, gamma, beta)
    embeds = out.reshape(token_indices.shape + (_D,))
    padding_mask = mask.reshape(token_indices.shape).astype(jnp.bool_)
    return embeds, padding_mask


# speedup vs baseline: 4.1434x; 2.5793x over previous
import functools

import jax
import jax.numpy as jnp
from jax import lax
from jax.experimental import pallas as pl
from jax.experimental.pallas import tpu as pltpu
from jax.experimental.pallas import tpu_sc as plsc

_D = 1024
_PAD = 1
_EPS = 1e-05
_EPS_FOLDED = _EPS / float(_D)

_NC = 2
_NS = 16
_L = 16
_NW = _NC * _NS
_B = 4 * 8192
_TPW = _B // _NW
_T = 16
_NCHUNK = _TPW // _T
_UNROLL = 8


def _rsqrt_vec(x):
    i = plsc.bitcast(x, jnp.int32)
    i = jnp.int32(0x5F3759DF) - lax.shift_right_arithmetic(i, 1)
    y = plsc.bitcast(i, jnp.float32)
    for _ in range(3):
        y = y * (1.5 - 0.5 * x * y * y)
    return y


_MESH = plsc.VectorSubcoreMesh(core_axis_name="c", subcore_axis_name="s")


@functools.partial(
    pl.kernel,
    out_type=(
        jax.ShapeDtypeStruct((_B, _D), jnp.float32),
        jax.ShapeDtypeStruct((_B,), jnp.int32),
    ),
    mesh=_MESH,
    scratch_types=[
        pltpu.VMEM((_TPW,), jnp.int32),
        pltpu.VMEM((_T, _D), jnp.float32),
        pltpu.VMEM((_T, _D), jnp.float32),
        pltpu.VMEM((_T, _D), jnp.float32),
        pltpu.VMEM((_T, _D), jnp.float32),
        pltpu.VMEM((_D,), jnp.float32),
        pltpu.VMEM((_D,), jnp.float32),
        pltpu.VMEM((_TPW,), jnp.int32),
        pltpu.SemaphoreType.DMA,
        pltpu.SemaphoreType.DMA,
        pltpu.SemaphoreType.DMA,
        pltpu.SemaphoreType.DMA,
    ],
    compiler_params=pltpu.CompilerParams(use_tc_tiling_on_sc=False,
                                         needs_layout_passes=False),
)
def _frontend(idx_hbm, table_hbm, gamma_hbm, beta_hbm, out_hbm, mask_hbm,
              idx_v, rows_a, rows_b, sta_a, sta_b, gamma_v, beta_v, mask_v,
              gsem_a, gsem_b, osem_a, osem_b):
    wid = lax.axis_index("s") * _NC + lax.axis_index("c")
    base = wid * _TPW

    pltpu.sync_copy(idx_hbm.at[pl.ds(base, _TPW)], idx_v)
    pltpu.sync_copy(gamma_hbm, gamma_v)
    pltpu.sync_copy(beta_hbm, beta_v)

    def mask_body(i, carry):
        v = idx_v[pl.ds(i * _L, _L)]
        mask_v[pl.ds(i * _L, _L)] = jnp.where(v == _PAD, jnp.int32(1),
                                              jnp.int32(0))
        return carry
    lax.fori_loop(0, _TPW // _L, mask_body, 0)
    pltpu.sync_copy(mask_v, mask_hbm.at[pl.ds(base, _TPW)])

    zeros = jnp.zeros((_L,), jnp.float32)
    lanes = lax.iota(jnp.int32, _L)
    shufs = [lanes ^ d for d in (8, 4, 2, 1)]

    def start_gather(c, rows, gsem):
        pltpu.async_copy(table_hbm.at[idx_v.at[pl.ds(c * _T, _T)]],
                         rows, gsem)

    def wait_gather(rows, gsem):
        pltpu.make_async_copy(table_hbm.at[pl.ds(0, _T)], rows, gsem).wait()

    def start_out(c, sta, osem):
        pltpu.async_copy(sta, out_hbm.at[pl.ds(base + c * _T, _T)], osem)

    def wait_out(sta, osem):
        pltpu.make_async_copy(sta, out_hbm.at[pl.ds(base, _T)], osem).wait()

    def compute(rows, sta):
        nacc = 4
        rinvs = []
        mrs = []
        for t in range(_T):
            init = tuple(zeros for _ in range(2 * nacc))

            def p1(jj, carry, t=t):
                accs = list(carry)
                for k in range(16):
                    v = rows[t, pl.ds(jj * 256 + k * _L, _L)]
                    accs[k % nacc] = accs[k % nacc] + v
                    accs[nacc + k % nacc] = accs[nacc + k % nacc] + v * v
                return tuple(accs)
            fin = plsc.parallel_loop(0, _D // 256, 1, carry=init)(p1)
            s = (fin[0] + fin[1]) + (fin[2] + fin[3])
            q = (fin[4] + fin[5]) + (fin[6] + fin[7])
            for sh in shufs:
                s = s + s.at[sh].get(mode="promise_in_bounds")
                q = q + q.at[sh].get(mode="promise_in_bounds")
            mean = s * (1.0 / _D)
            var = q * (1.0 / _D) - mean * mean
            rinv = _rsqrt_vec(var + _EPS_FOLDED)
            rinvs.append(rinv)
            mrs.append(mean * rinv)

        def p2(j):
            off = j * _L
            gv = gamma_v[pl.ds(off, _L)]
            bv = beta_v[pl.ds(off, _L)]
            for t in range(_T):
                v = rows[t, pl.ds(off, _L)]
                sta[t, pl.ds(off, _L)] = (v * rinvs[t] - mrs[t]) * gv + bv
        plsc.parallel_loop(0, _D // _L, 1)(p2)

    start_gather(0, rows_a, gsem_a)

    def pair_body(cp, carry):
        c0 = 2 * cp
        c1 = c0 + 1
        start_gather(c1, rows_b, gsem_b)
        wait_gather(rows_a, gsem_a)

        @pl.when(cp > 0)
        def _():
            wait_out(sta_a, osem_a)
        compute(rows_a, sta_a)
        start_out(c0, sta_a, osem_a)

        @pl.when(cp < _NCHUNK // 2 - 1)
        def _():
            start_gather(c0 + 2, rows_a, gsem_a)
        wait_gather(rows_b, gsem_b)

        @pl.when(cp > 0)
        def _():
            wait_out(sta_b, osem_b)
        compute(rows_b, sta_b)
        start_out(c1, sta_b, osem_b)
        return carry
    lax.fori_loop(0, _NCHUNK // 2, pair_body, 0)

    wait_out(sta_a, osem_a)
    wait_out(sta_b, osem_b)


def kernel(token_indices, embed_table, gamma, beta):
    idx_flat = token_indices.reshape(-1)
    out, mask = _frontend(idx_flat, embed_table, gamma, beta)
    embeds = out.reshape(token_indices.shape + (_D,))
    padding_mask = mask.reshape(token_indices.shape).astype(jnp.bool_)
    return embeds, padding_mask


# speedup vs baseline: 11.7781x; 2.8426x over previous
import functools

import jax
import jax.numpy as jnp
from jax import lax
from jax.experimental import pallas as pl
from jax.experimental.pallas import tpu as pltpu
from jax.experimental.pallas import tpu_sc as plsc

_D = 1024
_PAD = 1
_EPS = 1e-05
_EPS_FOLDED = _EPS / float(_D)

_NC = 2
_NS = 16
_L = 16
_NW = _NC * _NS
_B = 4 * 8192
_TPW = _B // _NW
_T = 16
_NCHUNK = _TPW // _T
_UNROLL = 8


def _rsqrt_vec(x):
    i = plsc.bitcast(x, jnp.int32)
    i = jnp.int32(0x5F3759DF) - lax.shift_right_arithmetic(i, 1)
    y = plsc.bitcast(i, jnp.float32)
    for _ in range(3):
        y = y * (1.5 - 0.5 * x * y * y)
    return y


_MESH = plsc.VectorSubcoreMesh(core_axis_name="c", subcore_axis_name="s")


@functools.partial(
    pl.kernel,
    out_type=(
        jax.ShapeDtypeStruct((_B, _D), jnp.float32),
        jax.ShapeDtypeStruct((_B,), jnp.int32),
    ),
    mesh=_MESH,
    scratch_types=[
        pltpu.VMEM((_TPW,), jnp.int32),
        pltpu.VMEM((_T, _D), jnp.float32),
        pltpu.VMEM((_T, _D), jnp.float32),
        pltpu.VMEM((_T, _D), jnp.float32),
        pltpu.VMEM((_T, _D), jnp.float32),
        pltpu.VMEM((_D,), jnp.float32),
        pltpu.VMEM((_D,), jnp.float32),
        pltpu.VMEM((_TPW,), jnp.int32),
        pltpu.SemaphoreType.DMA,
        pltpu.SemaphoreType.DMA,
        pltpu.SemaphoreType.DMA,
        pltpu.SemaphoreType.DMA,
    ],
    compiler_params=pltpu.CompilerParams(use_tc_tiling_on_sc=True,
                                         needs_layout_passes=False),
)
def _frontend(idx_hbm, table_hbm, gamma_hbm, beta_hbm, out_hbm, mask_hbm,
              idx_v, rows_a, rows_b, sta_a, sta_b, gamma_v, beta_v, mask_v,
              gsem_a, gsem_b, osem_a, osem_b):
    wid = lax.axis_index("s") * _NC + lax.axis_index("c")
    base = wid * _TPW

    pltpu.sync_copy(idx_hbm.at[pl.ds(base, _TPW)], idx_v)
    pltpu.sync_copy(gamma_hbm, gamma_v)
    pltpu.sync_copy(beta_hbm, beta_v)

    def mask_body(i, carry):
        v = idx_v[pl.ds(i * _L, _L)]
        mask_v[pl.ds(i * _L, _L)] = jnp.where(v == _PAD, jnp.int32(1),
                                              jnp.int32(0))
        return carry
    lax.fori_loop(0, _TPW // _L, mask_body, 0)
    pltpu.sync_copy(mask_v, mask_hbm.at[pl.ds(base, _TPW)])

    zeros = jnp.zeros((_L,), jnp.float32)
    lanes = lax.iota(jnp.int32, _L)
    shufs = [lanes ^ d for d in (8, 4, 2, 1)]

    def start_gather(c, rows, gsem):
        pltpu.async_copy(table_hbm.at[idx_v.at[pl.ds(c * _T, _T)]],
                         rows, gsem)

    def wait_gather(rows, gsem):
        pltpu.make_async_copy(table_hbm.at[pl.ds(0, _T)], rows, gsem).wait()

    def start_out(c, sta, osem):
        pltpu.async_copy(sta, out_hbm.at[pl.ds(base + c * _T, _T)], osem)

    def wait_out(sta, osem):
        pltpu.make_async_copy(sta, out_hbm.at[pl.ds(base, _T)], osem).wait()

    def compute(rows, sta):
        nacc = 4
        rinvs = []
        mrs = []
        for t in range(_T):
            init = tuple(zeros for _ in range(2 * nacc))

            def p1(jj, carry, t=t):
                accs = list(carry)
                for k in range(16):
                    v = rows[t, pl.ds(jj * 256 + k * _L, _L)]
                    accs[k % nacc] = accs[k % nacc] + v
                    accs[nacc + k % nacc] = accs[nacc + k % nacc] + v * v
                return tuple(accs)
            fin = plsc.parallel_loop(0, _D // 256, 1, carry=init)(p1)
            s = (fin[0] + fin[1]) + (fin[2] + fin[3])
            q = (fin[4] + fin[5]) + (fin[6] + fin[7])
            for sh in shufs:
                s = s + s.at[sh].get(mode="promise_in_bounds")
                q = q + q.at[sh].get(mode="promise_in_bounds")
            mean = s * (1.0 / _D)
            var = q * (1.0 / _D) - mean * mean
            rinv = _rsqrt_vec(var + _EPS_FOLDED)
            rinvs.append(rinv)
            mrs.append(mean * rinv)

        def p2(j):
            off = j * _L
            gv = gamma_v[pl.ds(off, _L)]
            bv = beta_v[pl.ds(off, _L)]
            for t in range(_T):
                v = rows[t, pl.ds(off, _L)]
                sta[t, pl.ds(off, _L)] = (v * rinvs[t] - mrs[t]) * gv + bv
        plsc.parallel_loop(0, _D // _L, 1)(p2)

    start_gather(0, rows_a, gsem_a)

    def pair_body(cp, carry):
        c0 = 2 * cp
        c1 = c0 + 1
        start_gather(c1, rows_b, gsem_b)
        wait_gather(rows_a, gsem_a)

        @pl.when(cp > 0)
        def _():
            wait_out(sta_a, osem_a)
        compute(rows_a, sta_a)
        start_out(c0, sta_a, osem_a)

        @pl.when(cp < _NCHUNK // 2 - 1)
        def _():
            start_gather(c0 + 2, rows_a, gsem_a)
        wait_gather(rows_b, gsem_b)

        @pl.when(cp > 0)
        def _():
            wait_out(sta_b, osem_b)
        compute(rows_b, sta_b)
        start_out(c1, sta_b, osem_b)
        return carry
    lax.fori_loop(0, _NCHUNK // 2, pair_body, 0)

    wait_out(sta_a, osem_a)
    wait_out(sta_b, osem_b)


def kernel(token_indices, embed_table, gamma, beta):
    idx_flat = token_indices.reshape(-1)
    out, mask = _frontend(idx_flat, embed_table, gamma, beta)
    embeds = out.reshape(token_indices.shape + (_D,))
    padding_mask = mask.reshape(token_indices.shape).astype(jnp.bool_)
    return embeds, padding_mask
